# Initial kernel scaffold; baseline (speedup 1.0000x reference)
import functools

import jax
import jax.numpy as jnp
from jax import lax
from jax.experimental import pallas as pl
from jax.experimental.pallas import tpu as pltpu, tpu_sc as plsc

N = 10000
E = 320000
D = 128
NC = 2
NS = 16
NW = NC * NS
EPW = E // NW
CH = 80
NCHUNK = EPW // CH
RPS = N // NS
BS = 1000

_HI = jax.lax.Precision.HIGHEST


def _dot_t(a, w):
    return jax.lax.dot_general(a, w, (((1,), (1,)), ((), ())),
                               preferred_element_type=jnp.float32,
                               precision=_HI)


def _sc_body(with_cnt, x_hbm, src_hbm, dst_hbm, zf_hbm, zc_hbm,
             p_hbm, cp_hbm, sidx_v, didx_v, rows_v, ones_v, sem,
             acc_sh, cacc_sh):
    c = lax.axis_index("c")
    s = lax.axis_index("s")
    wid = c * NS + s

    pltpu.sync_copy(zf_hbm, acc_sh.at[pl.ds(s * RPS, RPS)])
    if with_cnt:
        pltpu.sync_copy(zc_hbm, cacc_sh.at[pl.ds(s * RPS, RPS)])

        def _ones(i, carry):
            ones_v[i, :] = jnp.ones((16,), jnp.float32)
            return carry
        lax.fori_loop(0, CH, _ones, 0)
    plsc.subcore_barrier()

    pltpu.sync_copy(src_hbm.at[pl.ds(wid * NCHUNK, NCHUNK)], sidx_v)
    pltpu.sync_copy(dst_hbm.at[pl.ds(wid * NCHUNK, NCHUNK)], didx_v)

    def _chunk(j, carry):
        pltpu.async_copy(x_hbm.at[sidx_v.at[j]], rows_v, sem).wait()
        pltpu.sync_copy(rows_v, acc_sh.at[didx_v.at[j]], add=True)
        if with_cnt:
            pltpu.sync_copy(ones_v, cacc_sh.at[didx_v.at[j]], add=True)
        return carry
    lax.fori_loop(0, NCHUNK, _chunk, 0)
    plsc.subcore_barrier()

    pltpu.sync_copy(acc_sh.at[pl.ds(s * RPS, RPS)],
                    p_hbm.at[pl.ds(c * N + s * RPS, RPS)])
    if with_cnt:
        pltpu.sync_copy(cacc_sh.at[pl.ds(s * RPS, RPS)],
                        cp_hbm.at[pl.ds(c * N + s * RPS, RPS)])


def _make_sc_pass(with_cnt):
    mesh = plsc.VectorSubcoreMesh(core_axis_name="c", subcore_axis_name="s")
    out_type = [jax.ShapeDtypeStruct((NC * N, D), jnp.float32)]
    if with_cnt:
        out_type.append(jax.ShapeDtypeStruct((NC * N, 16), jnp.float32))
    scratch = [
        pltpu.VMEM((NCHUNK, CH), jnp.int32),
        pltpu.VMEM((NCHUNK, CH), jnp.int32),
        pltpu.VMEM((CH, D), jnp.float32),
        pltpu.VMEM((CH, 16), jnp.float32),
        pltpu.SemaphoreType.DMA,
        pltpu.VMEM_SHARED((N, D), jnp.float32),
        pltpu.VMEM_SHARED((N, 16), jnp.float32),
    ]

    def body(*refs):
        if with_cnt:
            (x, src, dst, zf, zc, p, cp, si, di, rows, ones, sem, acc,
             cacc) = refs
        else:
            (x, src, dst, zf, zc, p, si, di, rows, ones, sem, acc,
             cacc) = refs
            cp = None
        _sc_body(with_cnt, x, src, dst, zf, zc, p, cp, si, di, rows, ones,
                 sem, acc, cacc)

    return pl.kernel(body, out_type=tuple(out_type), mesh=mesh,
                     scratch_types=scratch)


_sc_pass_cnt = _make_sc_pass(True)
_sc_pass = _make_sc_pass(False)


def _layer1_body(h_ref, pt_ref, pb_ref, ct_ref, cb_ref, wl_ref, bl_ref,
                 wr_ref, out_ref, inv_ref):
    cnt = ct_ref[:, 0:1] + cb_ref[:, 0:1]
    inv = 1.0 / jnp.maximum(cnt, 1.0)
    inv_ref[...] = jnp.broadcast_to(inv, (BS, 16))
    agg = (pt_ref[...] + pb_ref[...]) * inv
    acc = _dot_t(agg, wl_ref[...]) + _dot_t(h_ref[...], wr_ref[...])
    acc = acc + bl_ref[...]
    out_ref[...] = jnp.maximum(acc, 0.0)


def _layerN_body(relu, h_ref, pt_ref, pb_ref, inv_ref, wl_ref, bl_ref,
                 wr_ref, out_ref):
    inv = inv_ref[:, 0:1]
    agg = (pt_ref[...] + pb_ref[...]) * inv
    acc = _dot_t(agg, wl_ref[...]) + _dot_t(h_ref[...], wr_ref[...])
    acc = acc + bl_ref[...]
    out_ref[...] = jnp.maximum(acc, 0.0) if relu else acc


def _row_spec(w):
    return pl.BlockSpec((BS, w), lambda i: (i, 0))


def _half_spec(w, lo):
    off = (lo * N) // BS
    return pl.BlockSpec((BS, w), lambda i, o=off: (i + o, 0))


def _w_spec(r, c):
    return pl.BlockSpec((r, c), lambda i: (0, 0))


def _tc_layer1(h, p, cp, wl, bl, wr):
    grid = (N // BS,)
    return pl.pallas_call(
        _layer1_body,
        grid=grid,
        in_specs=[
            _row_spec(D), _half_spec(D, 0), _half_spec(D, 1),
            _half_spec(16, 0), _half_spec(16, 1),
            _w_spec(D, D), _w_spec(1, D), _w_spec(D, D),
        ],
        out_specs=[_row_spec(D), _row_spec(16)],
        out_shape=[jax.ShapeDtypeStruct((N, D), jnp.float32),
                   jax.ShapeDtypeStruct((N, 16), jnp.float32)],
    )(h, p, p, cp, cp, wl, bl.reshape(1, D), wr)


def _tc_layerN(h, p, inv, wl, bl, wr, relu):
    grid = (N // BS,)
    return pl.pallas_call(
        functools.partial(_layerN_body, relu),
        grid=grid,
        in_specs=[
            _row_spec(D), _half_spec(D, 0), _half_spec(D, 1), _row_spec(16),
            _w_spec(D, D), _w_spec(1, D), _w_spec(D, D),
        ],
        out_specs=_row_spec(D),
        out_shape=jax.ShapeDtypeStruct((N, D), jnp.float32),
    )(h, p, p, inv, wl, bl.reshape(1, D), wr)


def _heads_body(h_ref, p1w_ref, p1b_ref, p2w_ref, p2b_ref, n1h_ref, n1z_ref,
                n1b_ref, n2w_ref, n2b_ref, hn_ref, zn_ref, l_ref):
    h = h_ref[...]
    z1 = jnp.maximum(_dot_t(h, p1w_ref[...]) + p1b_ref[...], 0.0)
    z = _dot_t(z1, p2w_ref[...]) + p2b_ref[...]
    zn = z / jnp.maximum(
        jnp.sqrt(jnp.sum(z * z, axis=1, keepdims=True)), 1e-12)
    u = jnp.maximum(
        _dot_t(h, n1h_ref[...]) + _dot_t(zn, n1z_ref[...]) + n1b_ref[...],
        0.0)
    l_ref[...] = _dot_t(u, n2w_ref[...]) + n2b_ref[...]
    hn_ref[...] = h / jnp.maximum(
        jnp.sqrt(jnp.sum(h * h, axis=1, keepdims=True)), 1e-12)
    zn_ref[...] = zn


def _tc_heads(hcat, p1_W, p1_b, p2_W, p2_b, n1_W, n1_b, n2_W, n2_b):
    P = p1_W.shape[0]
    grid = ((2 * N) // BS,)
    n1h = n1_W[:, :D]
    n1z = n1_W[:, D:]
    return pl.pallas_call(
        _heads_body,
        grid=grid,
        in_specs=[
            _row_spec(D),
            _w_spec(P, D), _w_spec(1, P), _w_spec(P, P), _w_spec(1, P),
            _w_spec(P, D), _w_spec(P, P), _w_spec(1, P),
            _w_spec(1, P), _w_spec(1, 1),
        ],
        out_specs=[_row_spec(D), _row_spec(P), _row_spec(1)],
        out_shape=[jax.ShapeDtypeStruct((2 * N, D), jnp.float32),
                   jax.ShapeDtypeStruct((2 * N, P), jnp.float32),
                   jax.ShapeDtypeStruct((2 * N, 1), jnp.float32)],
    )(hcat, p1_W, p1_b.reshape(1, P), p2_W, p2_b.reshape(1, P),
      n1h, n1z, n1_b.reshape(1, P), n2_W, n2_b.reshape(1, 1))


def _encode(x, src2d, dst2d, zf, zc, g1_Wl, g1_bl, g1_Wr, g2_Wl, g2_bl,
            g2_Wr, g3_Wl, g3_bl, g3_Wr):
    p1, cp = _sc_pass_cnt(x, src2d, dst2d, zf, zc)
    h1, inv = _tc_layer1(x, p1, cp, g1_Wl, g1_bl, g1_Wr)
    (p2,) = _sc_pass(h1, src2d, dst2d, zf, zc)
    h2 = _tc_layerN(h1, p2, inv, g2_Wl, g2_bl, g2_Wr, relu=True)
    (p3,) = _sc_pass(h2, src2d, dst2d, zf, zc)
    return _tc_layerN(h2, p3, inv, g3_Wl, g3_bl, g3_Wr, relu=False)


def kernel(xA, xB, edge_index_A, edge_index_B, g1_Wl, g1_bl, g1_Wr,
           g2_Wl, g2_bl, g2_Wr, g3_Wl, g3_bl, g3_Wr,
           p1_W, p1_b, p2_W, p2_b, n1_W, n1_b, n2_W, n2_b):
    srcA = edge_index_A[0].reshape(NW * NCHUNK, CH)
    dstA = edge_index_A[1].reshape(NW * NCHUNK, CH)
    srcB = edge_index_B[0].reshape(NW * NCHUNK, CH)
    dstB = edge_index_B[1].reshape(NW * NCHUNK, CH)
    zf = jnp.zeros((RPS, D), jnp.float32)
    zc = jnp.zeros((RPS, 16), jnp.float32)

    g = (g1_Wl, g1_bl, g1_Wr, g2_Wl, g2_bl, g2_Wr, g3_Wl, g3_bl, g3_Wr)
    hA = _encode(xA, srcA, dstA, zf, zc, *g)
    hB = _encode(xB, srcB, dstB, zf, zc, *g)

    hcat = jnp.concatenate([hA, hB], axis=0)
    hn, zn, l = _tc_heads(hcat, p1_W, p1_b, p2_W, p2_b, n1_W, n1_b,
                          n2_W, n2_b)
    return (hn[:N], hn[N:], zn[:N], zn[N:], l[:N, 0], l[N:, 0])


# speedup vs baseline: 4.5886x; 4.5886x over previous
import functools

import jax
import jax.numpy as jnp
from jax import lax
from jax.experimental import pallas as pl
from jax.experimental.pallas import tpu as pltpu, tpu_sc as plsc

N = 10000
E = 320000
D = 128
NC = 2
NS = 16
NW = NC * NS
EPW = E // NW
CH = 80
NCHUNK = EPW // CH
RPS = 624
REM = N - NS * RPS
BS = 1000

_HI = jax.lax.Precision.DEFAULT


def _dot_t(a, w):
    return jax.lax.dot_general(a, w, (((1,), (1,)), ((), ())),
                               preferred_element_type=jnp.float32,
                               precision=_HI)


def _stripe_copy(src_fn, dst_fn, s):
    pltpu.sync_copy(src_fn(0, RPS, s * RPS), dst_fn(0, RPS, s * RPS))

    @pl.when(s == NS - 1)
    def _rem():
        pltpu.sync_copy(src_fn(1, REM, NS * RPS), dst_fn(1, REM, NS * RPS))


def _sc_agg_body(x_hbm, src_hbm, dst_hbm, zf_hbm, p_hbm,
                 sidx_v, didx_v, rows_v, sem, acc_sh):
    c = lax.axis_index("c")
    s = lax.axis_index("s")
    wid = c * NS + s

    _stripe_copy(lambda r, n, o: zf_hbm.at[pl.ds(0, n)],
                 lambda r, n, o: acc_sh.at[pl.ds(o, n)], s)
    plsc.subcore_barrier()

    pltpu.sync_copy(src_hbm.at[wid], sidx_v)
    pltpu.sync_copy(dst_hbm.at[wid], didx_v)

    def _chunk(j, carry):
        pltpu.async_copy(x_hbm.at[sidx_v.at[j]], rows_v, sem).wait()
        pltpu.sync_copy(rows_v, acc_sh.at[didx_v.at[j]], add=True)
        return carry
    lax.fori_loop(0, NCHUNK, _chunk, 0)
    plsc.subcore_barrier()

    _stripe_copy(lambda r, n, o: acc_sh.at[pl.ds(o, n)],
                 lambda r, n, o: p_hbm.at[pl.ds(c * N + o, n)], s)


def _sc_cnt_body(dst_hbm, zc_hbm, ones_hbm, cp_hbm, didx_v, ones_v, cacc_sh):
    c = lax.axis_index("c")
    s = lax.axis_index("s")
    wid = c * NS + s

    _stripe_copy(lambda r, n, o: zc_hbm.at[pl.ds(0, n)],
                 lambda r, n, o: cacc_sh.at[pl.ds(o, n)], s)
    pltpu.sync_copy(ones_hbm, ones_v)
    plsc.subcore_barrier()

    pltpu.sync_copy(dst_hbm.at[wid], didx_v)

    def _chunk(j, carry):
        pltpu.sync_copy(ones_v, cacc_sh.at[didx_v.at[j]], add=True)
        return carry
    lax.fori_loop(0, NCHUNK, _chunk, 0)
    plsc.subcore_barrier()

    _stripe_copy(lambda r, n, o: cacc_sh.at[pl.ds(o, n)],
                 lambda r, n, o: cp_hbm.at[pl.ds(c * N + o, n)], s)


@functools.lru_cache(maxsize=None)
def _sc_kernels():
    mesh = plsc.VectorSubcoreMesh(core_axis_name="c", subcore_axis_name="s",
                                  num_cores=NC, num_subcores=NS)
    sc_pass = pl.kernel(
        _sc_agg_body,
        out_type=jax.ShapeDtypeStruct((NC * N, D), jnp.float32),
        mesh=mesh,
        scratch_types=[
            pltpu.VMEM((NCHUNK, CH), jnp.int32),
            pltpu.VMEM((NCHUNK, CH), jnp.int32),
            pltpu.VMEM((CH, D), jnp.float32),
            pltpu.SemaphoreType.DMA,
            pltpu.VMEM_SHARED((N, D), jnp.float32),
        ])
    sc_cnt = pl.kernel(
        _sc_cnt_body,
        out_type=jax.ShapeDtypeStruct((NC * N, D), jnp.float32),
        mesh=mesh,
        scratch_types=[
            pltpu.VMEM((NCHUNK, CH), jnp.int32),
            pltpu.VMEM((CH, D), jnp.float32),
            pltpu.VMEM_SHARED((N, D), jnp.float32),
        ])
    return sc_pass, sc_cnt


def _sc_pass(x, src2d, dst2d, zf):
    return _sc_kernels()[0](x, src2d, dst2d, zf)


def _sc_cnt(dst2d, zc, ones):
    return _sc_kernels()[1](dst2d, zc, ones)


def _layer1_body(h_ref, pt_ref, pb_ref, ct_ref, cb_ref, wl_ref, bl_ref,
                 wr_ref, out_ref, inv_ref):
    cnt = ct_ref[:, 0:1] + cb_ref[:, 0:1]
    inv = 1.0 / jnp.maximum(cnt, 1.0)
    inv_ref[...] = jnp.broadcast_to(inv, (BS, 16))
    agg = (pt_ref[...] + pb_ref[...]) * inv
    acc = _dot_t(agg, wl_ref[...]) + _dot_t(h_ref[...], wr_ref[...])
    acc = acc + bl_ref[...]
    out_ref[...] = jnp.maximum(acc, 0.0)


def _layerN_body(relu, h_ref, pt_ref, pb_ref, inv_ref, wl_ref, bl_ref,
                 wr_ref, out_ref):
    inv = inv_ref[:, 0:1]
    agg = (pt_ref[...] + pb_ref[...]) * inv
    acc = _dot_t(agg, wl_ref[...]) + _dot_t(h_ref[...], wr_ref[...])
    acc = acc + bl_ref[...]
    out_ref[...] = jnp.maximum(acc, 0.0) if relu else acc


def _row_spec(w):
    return pl.BlockSpec((BS, w), lambda i: (i, 0))


def _half_spec(w, lo):
    off = (lo * N) // BS
    return pl.BlockSpec((BS, w), lambda i, o=off: (i + o, 0))


def _w_spec(r, c):
    return pl.BlockSpec((r, c), lambda i: (0, 0))


def _tc_layer1(h, p, cp, wl, bl, wr):
    grid = (N // BS,)
    return pl.pallas_call(
        _layer1_body,
        grid=grid,
        in_specs=[
            _row_spec(D), _half_spec(D, 0), _half_spec(D, 1),
            _half_spec(D, 0), _half_spec(D, 1),
            _w_spec(D, D), _w_spec(1, D), _w_spec(D, D),
        ],
        out_specs=[_row_spec(D), _row_spec(16)],
        out_shape=[jax.ShapeDtypeStruct((N, D), jnp.float32),
                   jax.ShapeDtypeStruct((N, 16), jnp.float32)],
    )(h, p, p, cp, cp, wl, bl.reshape(1, D), wr)


def _tc_layerN(h, p, inv, wl, bl, wr, relu):
    grid = (N // BS,)
    return pl.pallas_call(
        functools.partial(_layerN_body, relu),
        grid=grid,
        in_specs=[
            _row_spec(D), _half_spec(D, 0), _half_spec(D, 1), _row_spec(16),
            _w_spec(D, D), _w_spec(1, D), _w_spec(D, D),
        ],
        out_specs=_row_spec(D),
        out_shape=jax.ShapeDtypeStruct((N, D), jnp.float32),
    )(h, p, p, inv, wl, bl.reshape(1, D), wr)


def _heads_body(h_ref, p1w_ref, p1b_ref, p2w_ref, p2b_ref, n1h_ref, n1z_ref,
                n1b_ref, n2w_ref, n2b_ref, hn_ref, zn_ref, l_ref):
    h = h_ref[...]
    z1 = jnp.maximum(_dot_t(h, p1w_ref[...]) + p1b_ref[...], 0.0)
    z = _dot_t(z1, p2w_ref[...]) + p2b_ref[...]
    zn = z / jnp.maximum(
        jnp.sqrt(jnp.sum(z * z, axis=1, keepdims=True)), 1e-12)
    u = jnp.maximum(
        _dot_t(h, n1h_ref[...]) + _dot_t(zn, n1z_ref[...]) + n1b_ref[...],
        0.0)
    l_ref[...] = (jnp.sum(u * n2w_ref[...], axis=1, keepdims=True)
                  + n2b_ref[0, 0])
    hn_ref[...] = h / jnp.maximum(
        jnp.sqrt(jnp.sum(h * h, axis=1, keepdims=True)), 1e-12)
    zn_ref[...] = zn


def _tc_heads(hcat, p1_W, p1_b, p2_W, p2_b, n1_W, n1_b, n2_W, n2_b):
    P = p1_W.shape[0]
    grid = ((2 * N) // BS,)
    n1h = n1_W[:, :D]
    n1z = n1_W[:, D:]
    return pl.pallas_call(
        _heads_body,
        grid=grid,
        in_specs=[
            _row_spec(D),
            _w_spec(P, D), _w_spec(1, P), _w_spec(P, P), _w_spec(1, P),
            _w_spec(P, D), _w_spec(P, P), _w_spec(1, P),
            _w_spec(1, P), _w_spec(1, 1),
        ],
        out_specs=[_row_spec(D), _row_spec(P), _row_spec(1)],
        out_shape=[jax.ShapeDtypeStruct((2 * N, D), jnp.float32),
                   jax.ShapeDtypeStruct((2 * N, P), jnp.float32),
                   jax.ShapeDtypeStruct((2 * N, 1), jnp.float32)],
    )(hcat, p1_W, p1_b.reshape(1, P), p2_W, p2_b.reshape(1, P),
      n1h, n1z, n1_b.reshape(1, P), n2_W, n2_b.reshape(1, 1))


def _encode(x, src2d, dst2d, zf, ones, g1_Wl, g1_bl, g1_Wr, g2_Wl, g2_bl,
            g2_Wr, g3_Wl, g3_bl, g3_Wr):
    cp = _sc_cnt(dst2d, zf, ones)
    p1 = _sc_pass(x, src2d, dst2d, zf)
    h1, inv = _tc_layer1(x, p1, cp, g1_Wl, g1_bl, g1_Wr)
    p2 = _sc_pass(h1, src2d, dst2d, zf)
    h2 = _tc_layerN(h1, p2, inv, g2_Wl, g2_bl, g2_Wr, relu=True)
    p3 = _sc_pass(h2, src2d, dst2d, zf)
    return _tc_layerN(h2, p3, inv, g3_Wl, g3_bl, g3_Wr, relu=False)


def kernel(xA, xB, edge_index_A, edge_index_B, g1_Wl, g1_bl, g1_Wr,
           g2_Wl, g2_bl, g2_Wr, g3_Wl, g3_bl, g3_Wr,
           p1_W, p1_b, p2_W, p2_b, n1_W, n1_b, n2_W, n2_b):
    srcA = edge_index_A[0].reshape(NW, NCHUNK, CH)
    dstA = edge_index_A[1].reshape(NW, NCHUNK, CH)
    srcB = edge_index_B[0].reshape(NW, NCHUNK, CH)
    dstB = edge_index_B[1].reshape(NW, NCHUNK, CH)
    zf = jnp.zeros((RPS, D), jnp.float32)
    ones = jnp.ones((CH, D), jnp.float32)

    g = (g1_Wl, g1_bl, g1_Wr, g2_Wl, g2_bl, g2_Wr, g3_Wl, g3_bl, g3_Wr)
    hA = _encode(xA, srcA, dstA, zf, ones, *g)
    hB = _encode(xB, srcB, dstB, zf, ones, *g)

    hcat = jnp.concatenate([hA, hB], axis=0)
    hn, zn, l = _tc_heads(hcat, p1_W, p1_b, p2_W, p2_b, n1_W, n1_b,
                          n2_W, n2_b)
    return (hn[:N], hn[N:], zn[:N], zn[N:], l[:N, 0], l[N:, 0])


# speedup vs baseline: 7.3078x; 1.5926x over previous
import functools

import jax
import jax.numpy as jnp
from jax import lax
from jax.experimental import pallas as pl
from jax.experimental.pallas import tpu as pltpu, tpu_sc as plsc

N = 10000
E = 320000
D = 128
NC = 2
NS = 16
NW = NC * NS
EPW = E // NW
CH = 100
NCHUNK = EPW // CH
RPS = 624
REM = N - NS * RPS
BS = 1000

_HI = jax.lax.Precision.DEFAULT


def _dot_t(a, w):
    return jax.lax.dot_general(a, w, (((1,), (1,)), ((), ())),
                               preferred_element_type=jnp.float32,
                               precision=_HI)


def _stripe_copy(src_fn, dst_fn, s):
    pltpu.sync_copy(src_fn(0, RPS, s * RPS), dst_fn(0, RPS, s * RPS))

    @pl.when(s == NS - 1)
    def _rem():
        pltpu.sync_copy(src_fn(1, REM, NS * RPS), dst_fn(1, REM, NS * RPS))


def _sc_agg_body(x_hbm, src_hbm, dst_hbm, zf_hbm, p_hbm,
                 sidx_v, dbuf0_v, dbuf1_v, rows0_v, rows1_v,
                 sem0, sem1, isem0, isem1, acc_sh):
    c = lax.axis_index("c")
    s = lax.axis_index("s")
    wid = c * NS + s

    _stripe_copy(lambda r, n, o: zf_hbm.at[pl.ds(0, n)],
                 lambda r, n, o: acc_sh.at[pl.ds(o, n)], s)
    plsc.subcore_barrier()

    pltpu.sync_copy(src_hbm.at[wid], sidx_v)

    def _g_start(j, buf, sem):
        pltpu.async_copy(x_hbm.at[sidx_v.at[j]], buf, sem)

    def _g_wait(j, buf, sem):
        pltpu.make_async_copy(x_hbm.at[sidx_v.at[j]], buf, sem).wait()

    def _i_start(j, buf, sem):
        pltpu.async_copy(dst_hbm.at[wid, j], buf, sem)

    def _i_wait(j, buf, sem):
        pltpu.make_async_copy(dst_hbm.at[wid, j], buf, sem).wait()

    pltpu.sync_copy(dst_hbm.at[wid, 0], dbuf0_v)
    _g_start(0, rows0_v, sem0)

    def _scatter(rows, dbuf):
        pltpu.sync_copy(rows, acc_sh.at[dbuf.at[0]], add=True)

    def _pair(j2, carry):
        e = 2 * j2
        _g_start(e + 1, rows1_v, sem1)
        _i_start(e + 1, dbuf1_v, isem1)
        _g_wait(e, rows0_v, sem0)
        _scatter(rows0_v, dbuf0_v)
        _g_start(e + 2, rows0_v, sem0)
        _i_start(e + 2, dbuf0_v, isem0)
        _g_wait(e + 1, rows1_v, sem1)
        _i_wait(e + 1, dbuf1_v, isem1)
        _scatter(rows1_v, dbuf1_v)
        _i_wait(e + 2, dbuf0_v, isem0)
        return carry
    lax.fori_loop(0, NCHUNK // 2 - 1, _pair, 0)

    e = NCHUNK - 2
    _g_start(e + 1, rows1_v, sem1)
    _i_start(e + 1, dbuf1_v, isem1)
    _g_wait(e, rows0_v, sem0)
    _scatter(rows0_v, dbuf0_v)
    _g_wait(e + 1, rows1_v, sem1)
    _i_wait(e + 1, dbuf1_v, isem1)
    _scatter(rows1_v, dbuf1_v)
    plsc.subcore_barrier()

    _stripe_copy(lambda r, n, o: acc_sh.at[pl.ds(o, n)],
                 lambda r, n, o: p_hbm.at[pl.ds(c * N + o, n)], s)


def _sc_cnt_body(dst_hbm, zc_hbm, ones_hbm, cp_hbm, didx_v, ones_v, cacc_sh):
    c = lax.axis_index("c")
    s = lax.axis_index("s")
    wid = c * NS + s

    _stripe_copy(lambda r, n, o: zc_hbm.at[pl.ds(0, n)],
                 lambda r, n, o: cacc_sh.at[pl.ds(o, n)], s)
    pltpu.sync_copy(ones_hbm, ones_v)
    plsc.subcore_barrier()

    pltpu.sync_copy(dst_hbm.at[wid], didx_v)

    def _chunk(j, carry):
        pltpu.sync_copy(ones_v, cacc_sh.at[didx_v.at[j, 0]], add=True)
        return carry
    lax.fori_loop(0, NCHUNK, _chunk, 0)
    plsc.subcore_barrier()

    _stripe_copy(lambda r, n, o: cacc_sh.at[pl.ds(o, n)],
                 lambda r, n, o: cp_hbm.at[pl.ds(c * N + o, n)], s)


@functools.lru_cache(maxsize=None)
def _sc_kernels():
    mesh = plsc.VectorSubcoreMesh(core_axis_name="c", subcore_axis_name="s",
                                  num_cores=NC, num_subcores=NS)
    sc_pass = pl.kernel(
        _sc_agg_body,
        out_type=jax.ShapeDtypeStruct((NC * N, D), jnp.float32),
        mesh=mesh,
        scratch_types=[
            pltpu.VMEM((NCHUNK, CH), jnp.int32),
            pltpu.VMEM((1, CH), jnp.int32),
            pltpu.VMEM((1, CH), jnp.int32),
            pltpu.VMEM((CH, D), jnp.float32),
            pltpu.VMEM((CH, D), jnp.float32),
            pltpu.SemaphoreType.DMA,
            pltpu.SemaphoreType.DMA,
            pltpu.SemaphoreType.DMA,
            pltpu.SemaphoreType.DMA,
            pltpu.VMEM_SHARED((N, D), jnp.float32),
        ])
    sc_cnt = pl.kernel(
        _sc_cnt_body,
        out_type=jax.ShapeDtypeStruct((NC * N, D), jnp.float32),
        mesh=mesh,
        scratch_types=[
            pltpu.VMEM((NCHUNK, 1, CH), jnp.int32),
            pltpu.VMEM((CH, D), jnp.float32),
            pltpu.VMEM_SHARED((N, D), jnp.float32),
        ])
    return sc_pass, sc_cnt


def _sc_pass(x, src2d, dst2d, zf):
    return _sc_kernels()[0](x, src2d, dst2d, zf)


def _sc_cnt(dst2d, zc, ones):
    return _sc_kernels()[1](dst2d, zc, ones)


def _layer1_body(h_ref, pt_ref, pb_ref, ct_ref, cb_ref, wl_ref, bl_ref,
                 wr_ref, out_ref, inv_ref):
    cnt = ct_ref[:, 0:1] + cb_ref[:, 0:1]
    inv = 1.0 / jnp.maximum(cnt, 1.0)
    inv_ref[...] = jnp.broadcast_to(inv, (BS, 16))
    agg = (pt_ref[...] + pb_ref[...]) * inv
    acc = _dot_t(agg, wl_ref[...]) + _dot_t(h_ref[...], wr_ref[...])
    acc = acc + bl_ref[...]
    out_ref[...] = jnp.maximum(acc, 0.0)


def _layerN_body(relu, h_ref, pt_ref, pb_ref, inv_ref, wl_ref, bl_ref,
                 wr_ref, out_ref):
    inv = inv_ref[:, 0:1]
    agg = (pt_ref[...] + pb_ref[...]) * inv
    acc = _dot_t(agg, wl_ref[...]) + _dot_t(h_ref[...], wr_ref[...])
    acc = acc + bl_ref[...]
    out_ref[...] = jnp.maximum(acc, 0.0) if relu else acc


def _row_spec(w):
    return pl.BlockSpec((BS, w), lambda i: (i, 0))


def _half_spec(w, lo):
    off = (lo * N) // BS
    return pl.BlockSpec((BS, w), lambda i, o=off: (i + o, 0))


def _w_spec(r, c):
    return pl.BlockSpec((r, c), lambda i: (0, 0))


def _tc_layer1(h, p, cp, wl, bl, wr):
    grid = (N // BS,)
    return pl.pallas_call(
        _layer1_body,
        grid=grid,
        in_specs=[
            _row_spec(D), _half_spec(D, 0), _half_spec(D, 1),
            _half_spec(D, 0), _half_spec(D, 1),
            _w_spec(D, D), _w_spec(1, D), _w_spec(D, D),
        ],
        out_specs=[_row_spec(D), _row_spec(16)],
        out_shape=[jax.ShapeDtypeStruct((N, D), jnp.float32),
                   jax.ShapeDtypeStruct((N, 16), jnp.float32)],
    )(h, p, p, cp, cp, wl, bl.reshape(1, D), wr)


def _tc_layerN(h, p, inv, wl, bl, wr, relu):
    grid = (N // BS,)
    return pl.pallas_call(
        functools.partial(_layerN_body, relu),
        grid=grid,
        in_specs=[
            _row_spec(D), _half_spec(D, 0), _half_spec(D, 1), _row_spec(16),
            _w_spec(D, D), _w_spec(1, D), _w_spec(D, D),
        ],
        out_specs=_row_spec(D),
        out_shape=jax.ShapeDtypeStruct((N, D), jnp.float32),
    )(h, p, p, inv, wl, bl.reshape(1, D), wr)


def _heads_body(h_ref, p1w_ref, p1b_ref, p2w_ref, p2b_ref, n1h_ref, n1z_ref,
                n1b_ref, n2w_ref, n2b_ref, hn_ref, zn_ref, l_ref):
    h = h_ref[...]
    z1 = jnp.maximum(_dot_t(h, p1w_ref[...]) + p1b_ref[...], 0.0)
    z = _dot_t(z1, p2w_ref[...]) + p2b_ref[...]
    zn = z / jnp.maximum(
        jnp.sqrt(jnp.sum(z * z, axis=1, keepdims=True)), 1e-12)
    u = jnp.maximum(
        _dot_t(h, n1h_ref[...]) + _dot_t(zn, n1z_ref[...]) + n1b_ref[...],
        0.0)
    l_ref[...] = (jnp.sum(u * n2w_ref[...], axis=1, keepdims=True)
                  + n2b_ref[0, 0])
    hn_ref[...] = h / jnp.maximum(
        jnp.sqrt(jnp.sum(h * h, axis=1, keepdims=True)), 1e-12)
    zn_ref[...] = zn


def _tc_heads(hcat, p1_W, p1_b, p2_W, p2_b, n1_W, n1_b, n2_W, n2_b):
    P = p1_W.shape[0]
    grid = ((2 * N) // BS,)
    n1h = n1_W[:, :D]
    n1z = n1_W[:, D:]
    return pl.pallas_call(
        _heads_body,
        grid=grid,
        in_specs=[
            _row_spec(D),
            _w_spec(P, D), _w_spec(1, P), _w_spec(P, P), _w_spec(1, P),
            _w_spec(P, D), _w_spec(P, P), _w_spec(1, P),
            _w_spec(1, P), _w_spec(1, 1),
        ],
        out_specs=[_row_spec(D), _row_spec(P), _row_spec(1)],
        out_shape=[jax.ShapeDtypeStruct((2 * N, D), jnp.float32),
                   jax.ShapeDtypeStruct((2 * N, P), jnp.float32),
                   jax.ShapeDtypeStruct((2 * N, 1), jnp.float32)],
    )(hcat, p1_W, p1_b.reshape(1, P), p2_W, p2_b.reshape(1, P),
      n1h, n1z, n1_b.reshape(1, P), n2_W, n2_b.reshape(1, 1))


def _encode(x, src2d, dst2d, zf, ones, g1_Wl, g1_bl, g1_Wr, g2_Wl, g2_bl,
            g2_Wr, g3_Wl, g3_bl, g3_Wr):
    cp = _sc_cnt(dst2d, zf, ones)
    p1 = _sc_pass(x, src2d, dst2d, zf)
    h1, inv = _tc_layer1(x, p1, cp, g1_Wl, g1_bl, g1_Wr)
    p2 = _sc_pass(h1, src2d, dst2d, zf)
    h2 = _tc_layerN(h1, p2, inv, g2_Wl, g2_bl, g2_Wr, relu=True)
    p3 = _sc_pass(h2, src2d, dst2d, zf)
    return _tc_layerN(h2, p3, inv, g3_Wl, g3_bl, g3_Wr, relu=False)


def kernel(xA, xB, edge_index_A, edge_index_B, g1_Wl, g1_bl, g1_Wr,
           g2_Wl, g2_bl, g2_Wr, g3_Wl, g3_bl, g3_Wr,
           p1_W, p1_b, p2_W, p2_b, n1_W, n1_b, n2_W, n2_b):
    srcA = edge_index_A[0].reshape(NW, NCHUNK, CH)
    dstA = edge_index_A[1].reshape(NW, NCHUNK, 1, CH)
    srcB = edge_index_B[0].reshape(NW, NCHUNK, CH)
    dstB = edge_index_B[1].reshape(NW, NCHUNK, 1, CH)
    zf = jnp.zeros((RPS, D), jnp.float32)
    ones = jnp.ones((CH, D), jnp.float32)

    g = (g1_Wl, g1_bl, g1_Wr, g2_Wl, g2_bl, g2_Wr, g3_Wl, g3_bl, g3_Wr)
    hA = _encode(xA, srcA, dstA, zf, ones, *g)
    hB = _encode(xB, srcB, dstB, zf, ones, *g)

    hcat = jnp.concatenate([hA, hB], axis=0)
    hn, zn, l = _tc_heads(hcat, p1_W, p1_b, p2_W, p2_b, n1_W, n1_b,
                          n2_W, n2_b)
    return (hn[:N], hn[N:], zn[:N], zn[N:], l[:N, 0], l[N:, 0])
